# baseline (speedup 1.0000x reference)
import functools

import jax
import jax.numpy as jnp
from jax import lax
from jax.experimental import pallas as pl
from jax.experimental.pallas import tpu as pltpu
from jax.experimental.pallas import tpu_sc as plsc

HID = 1024
EXPERTS = 8
TOPK = 2
ROWS = 8192
TM = 256
S = ROWS + EXPERTS * TM
NW = 32


def _routing(ids, wt):
    oh = (ids[:, None] == jnp.arange(EXPERTS, dtype=ids.dtype)[None, :]).astype(jnp.int32)
    csum = jnp.cumsum(oh, axis=0)
    counts = csum[-1]
    rank = jnp.take_along_axis(csum, ids[:, None], axis=1)[:, 0] - 1
    padded = ((counts + TM - 1) // TM) * TM
    ends = jnp.cumsum(padded)
    offsets = ends - padded
    pos = offsets[ids] + rank
    inv = jnp.zeros((S,), jnp.int32).at[pos].set(jnp.arange(ROWS, dtype=jnp.int32))
    wsort = jnp.zeros((S,), jnp.float32).at[pos].set(wt)
    tile_starts = jnp.arange(S // TM, dtype=jnp.int32) * TM
    tile_expert = jnp.minimum(
        jnp.searchsorted(ends, tile_starts, side="right"), EXPERTS - 1
    ).astype(jnp.int32)
    return pos, inv, wsort, tile_expert


_G_CH = 16
_G_NBUF = 3
_G_LANES = 2


def _sc_gather(table, idx):
    B = idx.shape[0]
    D = table.shape[1]
    b_per_w = B // NW
    b_per_lane = b_per_w // _G_LANES
    n_ch = b_per_lane // _G_CH
    mesh = plsc.VectorSubcoreMesh(core_axis_name="c", subcore_axis_name="s")

    @functools.partial(
        pl.kernel,
        mesh=mesh,
        out_type=jax.ShapeDtypeStruct((B, D), table.dtype),
        scratch_types=[
            pltpu.VMEM((b_per_w,), jnp.int32),
            [[pltpu.VMEM((_G_CH, D), table.dtype) for _ in range(_G_NBUF)]
             for _ in range(_G_LANES)],
            [pltpu.SemaphoreType.DMA for _ in range(_G_LANES)],
            [pltpu.SemaphoreType.DMA for _ in range(_G_LANES)],
        ],
    )
    def k(table_hbm, idx_hbm, out_hbm, idx_v, bufs, sems_g, sems_w):
        wid = lax.axis_index("s") * 2 + lax.axis_index("c")
        base = pl.multiple_of(wid * b_per_w, _G_CH)
        pltpu.sync_copy(idx_hbm.at[pl.ds(base, b_per_w)], idx_v)

        def start_gather(ln, c):
            off = ln * b_per_lane + c * _G_CH
            return pltpu.async_copy(
                table_hbm.at[idx_v.at[pl.ds(off, _G_CH)]],
                bufs[ln][c % _G_NBUF], sems_g[ln])

        gathers = {}
        writes = {}
        for c in range(min(_G_NBUF, n_ch)):
            for ln in range(_G_LANES):
                gathers[ln, c] = start_gather(ln, c)
        for c in range(n_ch):
            for ln in range(_G_LANES):
                gathers[ln, c].wait()
                writes[ln, c] = pltpu.async_copy(
                    bufs[ln][c % _G_NBUF],
                    out_hbm.at[pl.ds(base + ln * b_per_lane + c * _G_CH, _G_CH)],
                    sems_w[ln])
            nxt = c + _G_NBUF
            if nxt < n_ch:
                for ln in range(_G_LANES):
                    writes[ln, c].wait()
                    del writes[ln, c]
                    gathers[ln, nxt] = start_gather(ln, nxt)
        for key in list(writes):
            writes[key].wait()

    return k(table, idx)


_C_CH = 8
_C_NBUF = 3
_C_LANES = 2


def _sc_combine(y, pos):
    T = pos.shape[0] // 2
    D = y.shape[1]
    t_per_w = T // NW
    t_per_lane = t_per_w // _C_LANES
    n_ch = t_per_lane // _C_CH
    mesh = plsc.VectorSubcoreMesh(core_axis_name="c", subcore_axis_name="s")

    @functools.partial(
        pl.kernel,
        mesh=mesh,
        out_type=jax.ShapeDtypeStruct((T, D), jnp.float32),
        scratch_types=[
            pltpu.VMEM((2 * t_per_w,), jnp.int32),
            [[pltpu.VMEM((2 * _C_CH, D), jnp.float32) for _ in range(_C_NBUF)]
             for _ in range(_C_LANES)],
            [pltpu.SemaphoreType.DMA for _ in range(_C_LANES)],
            [pltpu.SemaphoreType.DMA for _ in range(_C_LANES)],
        ],
    )
    def k(y_hbm, pos_hbm, out_hbm, idx_v, bufs, sems_g, sems_w):
        wid = lax.axis_index("s") * 2 + lax.axis_index("c")
        base = pl.multiple_of(wid * t_per_w, _C_CH)
        pltpu.sync_copy(pos_hbm.at[pl.ds(2 * base, 2 * t_per_w)], idx_v)

        def start_gather(ln, c):
            off = 2 * (ln * t_per_lane + c * _C_CH)
            return pltpu.async_copy(
                y_hbm.at[idx_v.at[pl.ds(off, 2 * _C_CH)]],
                bufs[ln][c % _C_NBUF], sems_g[ln])

        gathers = {}
        writes = {}
        for c in range(min(_C_NBUF, n_ch)):
            for ln in range(_C_LANES):
                gathers[ln, c] = start_gather(ln, c)
        for c in range(n_ch):
            for ln in range(_C_LANES):
                gathers[ln, c].wait()
                buf = bufs[ln][c % _C_NBUF]

                def add_row(r, _, buf=buf):
                    for j in range(D // 16):
                        sl = pl.ds(j * 16, 16)
                        buf[r, sl] = buf[2 * r, sl] + buf[2 * r + 1, sl]
                    return ()

                lax.fori_loop(0, _C_CH, add_row, ())
                writes[ln, c] = pltpu.async_copy(
                    buf.at[pl.ds(0, _C_CH)],
                    out_hbm.at[pl.ds(base + ln * t_per_lane + c * _C_CH, _C_CH)],
                    sems_w[ln])
            nxt = c + _C_NBUF
            if nxt < n_ch:
                for ln in range(_C_LANES):
                    writes[ln, c].wait()
                    del writes[ln, c]
                    gathers[ln, nxt] = start_gather(ln, nxt)
        for key in list(writes):
            writes[key].wait()

    return k(y, pos)


def _gemm_body(te_ref, x_ref, w_ref, wv_ref, y_ref):
    x = x_ref[...].astype(jnp.bfloat16)
    y = jnp.dot(x, w_ref[0], preferred_element_type=jnp.float32)
    y_ref[...] = y * wv_ref[...]


def _grouped_gemm(x_sorted, w_bf, wsort, tile_expert):
    grid_spec = pltpu.PrefetchScalarGridSpec(
        num_scalar_prefetch=1,
        grid=(S // TM,),
        in_specs=[
            pl.BlockSpec((TM, HID), lambda i, te: (i, 0)),
            pl.BlockSpec((1, HID, HID), lambda i, te: (te[i], 0, 0)),
            pl.BlockSpec((TM, 1), lambda i, te: (i, 0)),
        ],
        out_specs=pl.BlockSpec((TM, HID), lambda i, te: (i, 0)),
    )
    return pl.pallas_call(
        _gemm_body,
        grid_spec=grid_spec,
        out_shape=jax.ShapeDtypeStruct((S, HID), jnp.float32),
    )(tile_expert, x_sorted, w_bf, wsort[:, None])


def kernel(intermediate_states, w, full_topk_ids, full_topk_weight):
    num_tokens = ROWS // TOPK
    ids = full_topk_ids[:num_tokens].reshape(-1)
    wt = full_topk_weight[:num_tokens].reshape(-1)

    pos, inv, wsort, tile_expert = _routing(ids, wt)

    x_sorted = _sc_gather(intermediate_states, inv)
    w_bf = w.astype(jnp.bfloat16)
    y_sorted = _grouped_gemm(x_sorted, w_bf, wsort, tile_expert)

    return _sc_combine(y_sorted, pos)


# speedup vs baseline: 1.4744x; 1.4744x over previous
import functools

import jax
import jax.numpy as jnp
from jax import lax
from jax.experimental import pallas as pl
from jax.experimental.pallas import tpu as pltpu
from jax.experimental.pallas import tpu_sc as plsc

HID = 1024
EXPERTS = 8
TOPK = 2
ROWS = 8192
TM = 256
S = ROWS + EXPERTS * TM
NW = 32


def _routing(ids):
    oh = (ids[:, None] == jnp.arange(EXPERTS, dtype=ids.dtype)[None, :]).astype(jnp.int32)
    csum = jnp.cumsum(oh, axis=0)
    counts = csum[-1]
    rank = jnp.take_along_axis(csum, ids[:, None], axis=1)[:, 0] - 1
    padded = ((counts + TM - 1) // TM) * TM
    ends = jnp.cumsum(padded)
    offsets = ends - padded
    pos = offsets[ids] + rank
    tile_starts = jnp.arange(S // TM, dtype=jnp.int32) * TM
    tile_expert = jnp.minimum(
        jnp.searchsorted(ends, tile_starts, side="right"), EXPERTS - 1
    ).astype(jnp.int32)
    return pos, tile_expert


_S_CH = 32
_S_NBUF = 3


def _sc_scatter_sort(x, pos3):
    D = x.shape[1]
    b_per_w = ROWS // NW
    n_ch = b_per_w // _S_CH
    mesh = plsc.VectorSubcoreMesh(core_axis_name="c", subcore_axis_name="s")

    @functools.partial(
        pl.kernel,
        mesh=mesh,
        out_type=jax.ShapeDtypeStruct((S, D), x.dtype),
        scratch_types=[
            pltpu.VMEM((n_ch, _S_CH), jnp.int32),
            [pltpu.VMEM((_S_CH, D), x.dtype) for _ in range(_S_NBUF)],
            pltpu.SemaphoreType.DMA,
            pltpu.SemaphoreType.DMA,
        ],
    )
    def k(x_hbm, pos_hbm, out_hbm, pos_v, bufs, sem_r, sem_w):
        wid = lax.axis_index("s") * 2 + lax.axis_index("c")
        base = pl.multiple_of(wid * b_per_w, _S_CH)
        pltpu.sync_copy(pos_hbm.at[wid], pos_v)

        def start_read(c):
            return pltpu.async_copy(
                x_hbm.at[pl.ds(base + c * _S_CH, _S_CH)],
                bufs[c % _S_NBUF], sem_r)

        reads = {}
        writes = {}
        for c in range(min(_S_NBUF, n_ch)):
            reads[c] = start_read(c)
        for c in range(n_ch):
            reads[c].wait()
            writes[c] = pltpu.async_copy(
                bufs[c % _S_NBUF], out_hbm.at[pos_v.at[c]], sem_w)
            nxt = c + _S_NBUF
            if nxt < n_ch:
                writes[c].wait()
                del writes[c]
                reads[nxt] = start_read(nxt)
        for c in list(writes):
            writes[c].wait()

    return k(x, pos3)


_C_CH = 8
_C_NBUF = 3
_C_LANES = 2


def _sc_combine(y, pos):
    T = pos.shape[0] // 2
    D = y.shape[1]
    t_per_w = T // NW
    t_per_lane = t_per_w // _C_LANES
    n_ch = t_per_lane // _C_CH
    mesh = plsc.VectorSubcoreMesh(core_axis_name="c", subcore_axis_name="s")

    @functools.partial(
        pl.kernel,
        mesh=mesh,
        out_type=jax.ShapeDtypeStruct((T, D), jnp.float32),
        scratch_types=[
            pltpu.VMEM((2 * t_per_w,), jnp.int32),
            [[pltpu.VMEM((2 * _C_CH, D), jnp.float32) for _ in range(_C_NBUF)]
             for _ in range(_C_LANES)],
            [pltpu.SemaphoreType.DMA for _ in range(_C_LANES)],
            [pltpu.SemaphoreType.DMA for _ in range(_C_LANES)],
        ],
    )
    def k(y_hbm, pos_hbm, out_hbm, idx_v, bufs, sems_g, sems_w):
        wid = lax.axis_index("s") * 2 + lax.axis_index("c")
        base = pl.multiple_of(wid * t_per_w, _C_CH)
        pltpu.sync_copy(pos_hbm.at[pl.ds(2 * base, 2 * t_per_w)], idx_v)

        def start_gather(ln, c):
            off = 2 * (ln * t_per_lane + c * _C_CH)
            return pltpu.async_copy(
                y_hbm.at[idx_v.at[pl.ds(off, 2 * _C_CH)]],
                bufs[ln][c % _C_NBUF], sems_g[ln])

        gathers = {}
        writes = {}
        for c in range(min(_C_NBUF, n_ch)):
            for ln in range(_C_LANES):
                gathers[ln, c] = start_gather(ln, c)
        for c in range(n_ch):
            for ln in range(_C_LANES):
                gathers[ln, c].wait()
                buf = bufs[ln][c % _C_NBUF]

                def add_row(r, _, buf=buf):
                    for j in range(D // 16):
                        sl = pl.ds(j * 16, 16)
                        buf[r, sl] = buf[2 * r, sl] + buf[2 * r + 1, sl]
                    return ()

                lax.fori_loop(0, _C_CH, add_row, ())
                writes[ln, c] = pltpu.async_copy(
                    buf.at[pl.ds(0, _C_CH)],
                    out_hbm.at[pl.ds(base + ln * t_per_lane + c * _C_CH, _C_CH)],
                    sems_w[ln])
            nxt = c + _C_NBUF
            if nxt < n_ch:
                for ln in range(_C_LANES):
                    writes[ln, c].wait()
                    del writes[ln, c]
                    gathers[ln, nxt] = start_gather(ln, nxt)
        for key in list(writes):
            writes[key].wait()

    return k(y, pos)


def _gemm_body(te_ref, x_ref, w_ref, y_ref):
    x = x_ref[...].astype(jnp.bfloat16)
    y_ref[...] = jnp.dot(x, w_ref[0], preferred_element_type=jnp.float32)


def _grouped_gemm(x_sorted, w_bf, tile_expert):
    grid_spec = pltpu.PrefetchScalarGridSpec(
        num_scalar_prefetch=1,
        grid=(S // TM,),
        in_specs=[
            pl.BlockSpec((TM, HID), lambda i, te: (i, 0)),
            pl.BlockSpec((1, HID, HID), lambda i, te: (te[i], 0, 0)),
        ],
        out_specs=pl.BlockSpec((TM, HID), lambda i, te: (i, 0)),
    )
    return pl.pallas_call(
        _gemm_body,
        grid_spec=grid_spec,
        out_shape=jax.ShapeDtypeStruct((S, HID), jnp.float32),
    )(tile_expert, x_sorted, w_bf)


def kernel(intermediate_states, w, full_topk_ids, full_topk_weight):
    num_tokens = ROWS // TOPK
    ids = full_topk_ids[:num_tokens].reshape(-1)
    wt = full_topk_weight[:num_tokens].reshape(-1)

    pos, tile_expert = _routing(ids)

    x_scaled = intermediate_states * wt[:, None]
    pos3 = pos.reshape(NW, (ROWS // NW) // _S_CH, _S_CH)
    x_sorted = _sc_scatter_sort(x_scaled, pos3)
    w_bf = w.astype(jnp.bfloat16)
    y_sorted = _grouped_gemm(x_sorted, w_bf, tile_expert)

    return _sc_combine(y_sorted, pos)
